# baseline (speedup 1.0000x reference)
import jax
import jax.numpy as jnp
from jax import lax
from jax.experimental import pallas as pl
from jax.experimental.pallas import tpu as pltpu
from jax.experimental.pallas import tpu_sc as plsc

N_NODES = 10000
N_PAD = 10240
D = 128
E = 320000
NSC = 16
LANES = 16
CHUNK = 80
VBATCH = 12
NVB = 21
EPS = NVB * VBATCH * CHUNK
E_PAD = NSC * EPS
BE = VBATCH * CHUNK
ROW_BLK = 64
RPS = N_PAD // NSC


def _mm_body(ux_ref, ix_ref, uw_ref, iw_ref, ou_ref, oi_ref):
    ou_ref[...] = jnp.dot(ux_ref[...], uw_ref[...],
                          preferred_element_type=jnp.float32)
    oi_ref[...] = jnp.dot(ix_ref[...], iw_ref[...],
                          preferred_element_type=jnp.float32)


def _project(user_x, item_x, user_weight, item_weight):
    n, d_in = user_x.shape
    d_out = user_weight.shape[1]
    blk = 1000
    return pl.pallas_call(
        _mm_body,
        grid=(n // blk,),
        in_specs=[
            pl.BlockSpec((blk, d_in), lambda i: (i, 0)),
            pl.BlockSpec((blk, d_in), lambda i: (i, 0)),
            pl.BlockSpec((d_in, d_out), lambda i: (0, 0)),
            pl.BlockSpec((d_in, d_out), lambda i: (0, 0)),
        ],
        out_specs=[
            pl.BlockSpec((blk, d_out), lambda i: (i, 0)),
            pl.BlockSpec((blk, d_out), lambda i: (i, 0)),
        ],
        out_shape=[
            jax.ShapeDtypeStruct((n, d_out), jnp.float32),
            jax.ShapeDtypeStruct((n, d_out), jnp.float32),
        ],
    )(user_x, item_x, user_weight, item_weight)


def _sc_body(xwu_hbm, xwi_hbm, rows_hbm, cols_hbm, vals_hbm,
             outu_hbm, outi_hbm,
             isrc0, isrc1, idst0, idst1, valb0, valb1,
             msga_v, msgb_v, msgc_v, acc_sh,
             gsem0, gsem1, gsem2, psem0, psem1):
    msg = (msga_v, msgb_v, msgc_v)
    gsem = (gsem0, gsem1, gsem2)
    isrc = (isrc0, isrc1)
    idst = (idst0, idst1)
    valb = (valb0, valb1)
    psem = (psem0, psem1)
    cid = lax.axis_index("c")
    sid = lax.axis_index("s")

    @pl.loop(0, ROW_BLK)
    def _(r):
        for b in range(D // LANES):
            msga_v.at[pl.ds(r, 1), pl.ds(b * LANES, LANES)][...] = (
                jnp.zeros((1, LANES), jnp.float32))
    for k in range(RPS // ROW_BLK):
        pltpu.sync_copy(msga_v.at[pl.ds(0, ROW_BLK)],
                        acc_sh.at[pl.ds(sid * RPS + k * ROW_BLK, ROW_BLK)])
    plsc.subcore_barrier()

    def scale(msg_v, vref, k):
        @pl.loop(0, CHUNK, step=LANES)
        def _(j0):
            val16 = vref[k, pl.ds(j0, LANES)]
            for u in range(LANES):
                srow = jnp.full((1, LANES), val16[u], jnp.float32)
                for b in range(D // LANES):
                    slc = (pl.ds(j0 + u, 1), pl.ds(b * LANES, LANES))
                    msg_v.at[slc][...] = msg_v.at[slc][...] * srow

    def prefetch(src_hbm, dst_hbm, t, s):
        pltpu.async_copy(src_hbm.at[sid, t], isrc[s], psem[s])
        pltpu.async_copy(dst_hbm.at[sid, t], idst[s], psem[s])
        pltpu.async_copy(vals_hbm.at[sid, t], valb[s], psem[s])

    def wait_isrc(src_hbm, t, s):
        pltpu.make_async_copy(src_hbm.at[sid, t], isrc[s], psem[s]).wait()

    def wait_dstval(dst_hbm, t, s):
        pltpu.make_async_copy(dst_hbm.at[sid, t], idst[s], psem[s]).wait()
        pltpu.make_async_copy(vals_hbm.at[sid, t], valb[s], psem[s]).wait()

    def run_batch(table_hbm, src_hbm, dst_hbm, t, t_next, s):
        o = (s + 1) % 2
        wait_dstval(dst_hbm, t, s)

        @pl.when(t_next < NVB)
        def _():
            prefetch(src_hbm, dst_hbm, t_next, o)
        for k in range(VBATCH):
            b = k % 3
            pltpu.make_async_copy(table_hbm.at[isrc[s].at[k]], msg[b],
                                  gsem[b]).wait()
            scale(msg[b], valb[s], k)
            if k + 2 < VBATCH:
                pltpu.async_copy(table_hbm.at[isrc[s].at[k + 2]],
                                 msg[(k + 2) % 3], gsem[(k + 2) % 3])
            else:
                kn = k + 2 - VBATCH
                @pl.when(t_next < NVB)
                def _():
                    if kn == 0:
                        wait_isrc(src_hbm, t_next, o)
                    pltpu.async_copy(table_hbm.at[isrc[o].at[kn]],
                                     msg[(k + 2) % 3], gsem[(k + 2) % 3])
            pltpu.sync_copy(msg[b], acc_sh.at[idst[s].at[k]], add=True)

    def accumulate(table_hbm, src_hbm, dst_hbm):
        prefetch(src_hbm, dst_hbm, 0, 0)
        wait_isrc(src_hbm, 0, 0)
        pltpu.async_copy(table_hbm.at[isrc[0].at[0]], msg[0], gsem[0])
        pltpu.async_copy(table_hbm.at[isrc[0].at[1]], msg[1], gsem[1])

        @pl.loop(0, (NVB + 1) // 2)
        def _(u):
            run_batch(table_hbm, src_hbm, dst_hbm, 2 * u, 2 * u + 1, 0)

            @pl.when(2 * u + 1 < NVB)
            def _():
                run_batch(table_hbm, src_hbm, dst_hbm, 2 * u + 1,
                          2 * u + 2, 1)

    @pl.when(cid == 0)
    def _():
        accumulate(xwi_hbm, cols_hbm, rows_hbm)

    @pl.when(cid == 1)
    def _():
        accumulate(xwu_hbm, rows_hbm, cols_hbm)

    plsc.subcore_barrier()

    def writeback(out_hbm):
        for k in range(RPS // ROW_BLK):
            r0 = sid * RPS + k * ROW_BLK
            pltpu.sync_copy(acc_sh.at[pl.ds(r0, ROW_BLK)],
                            msga_v.at[pl.ds(0, ROW_BLK)])

            @pl.loop(0, ROW_BLK)
            def _(r):
                for b in range(D // LANES):
                    slc = (pl.ds(r, 1), pl.ds(b * LANES, LANES))
                    msga_v.at[slc][...] = jnp.maximum(msga_v.at[slc][...],
                                                      0.0)
            pltpu.sync_copy(msga_v.at[pl.ds(0, ROW_BLK)],
                            out_hbm.at[pl.ds(r0, ROW_BLK)])

    @pl.when(cid == 0)
    def _():
        writeback(outu_hbm)

    @pl.when(cid == 1)
    def _():
        writeback(outi_hbm)


def kernel(user_x, item_x, ui_indices, ui_values, user_weight, item_weight):
    xw_user, xw_item = _project(user_x, item_x, user_weight, item_weight)
    pad = E_PAD - E
    rows = jnp.concatenate(
        [ui_indices[0].astype(jnp.int32), jnp.zeros((pad,), jnp.int32)])
    cols = jnp.concatenate(
        [ui_indices[1].astype(jnp.int32), jnp.zeros((pad,), jnp.int32)])
    vals = jnp.concatenate([ui_values, jnp.zeros((pad,), jnp.float32)])
    rows4 = rows.reshape(NSC, NVB, VBATCH, CHUNK)
    cols4 = cols.reshape(NSC, NVB, VBATCH, CHUNK)
    vals4 = vals.reshape(NSC, NVB, VBATCH, CHUNK)

    mesh = plsc.VectorSubcoreMesh(core_axis_name="c", subcore_axis_name="s")
    sc_fn = pl.kernel(
        _sc_body,
        out_type=[
            jax.ShapeDtypeStruct((N_PAD, D), jnp.float32),
            jax.ShapeDtypeStruct((N_PAD, D), jnp.float32),
        ],
        mesh=mesh,
        scratch_types=[
            pltpu.VMEM((VBATCH, CHUNK), jnp.int32),
            pltpu.VMEM((VBATCH, CHUNK), jnp.int32),
            pltpu.VMEM((VBATCH, CHUNK), jnp.int32),
            pltpu.VMEM((VBATCH, CHUNK), jnp.int32),
            pltpu.VMEM((VBATCH, CHUNK), jnp.float32),
            pltpu.VMEM((VBATCH, CHUNK), jnp.float32),
            pltpu.VMEM((CHUNK, D), jnp.float32),
            pltpu.VMEM((CHUNK, D), jnp.float32),
            pltpu.VMEM((CHUNK, D), jnp.float32),
            pltpu.VMEM_SHARED((N_PAD, D), jnp.float32),
            pltpu.SemaphoreType.DMA,
            pltpu.SemaphoreType.DMA,
            pltpu.SemaphoreType.DMA,
            pltpu.SemaphoreType.DMA,
            pltpu.SemaphoreType.DMA,
        ],
    )
    out_user, out_item = sc_fn(xw_user, xw_item, rows4, cols4, vals4)
    return (out_user[:N_NODES], out_item[:N_NODES])


# speedup vs baseline: 1.0117x; 1.0117x over previous
import jax
import jax.numpy as jnp
from jax import lax
from jax.experimental import pallas as pl
from jax.experimental.pallas import tpu as pltpu
from jax.experimental.pallas import tpu_sc as plsc

N_NODES = 10000
N_PAD = 10240
D = 128
E = 320000
NSC = 16
LANES = 16
CHUNK = 80
VBATCH = 9
NVB = 28
EPS = NVB * VBATCH * CHUNK
E_PAD = NSC * EPS
BE = VBATCH * CHUNK
ROW_BLK = 64
RPS = N_PAD // NSC


def _mm_body(ux_ref, ix_ref, uw_ref, iw_ref, ou_ref, oi_ref):
    ou_ref[...] = jnp.dot(ux_ref[...], uw_ref[...],
                          preferred_element_type=jnp.float32)
    oi_ref[...] = jnp.dot(ix_ref[...], iw_ref[...],
                          preferred_element_type=jnp.float32)


def _project(user_x, item_x, user_weight, item_weight):
    n, d_in = user_x.shape
    d_out = user_weight.shape[1]
    blk = 1000
    return pl.pallas_call(
        _mm_body,
        grid=(n // blk,),
        in_specs=[
            pl.BlockSpec((blk, d_in), lambda i: (i, 0)),
            pl.BlockSpec((blk, d_in), lambda i: (i, 0)),
            pl.BlockSpec((d_in, d_out), lambda i: (0, 0)),
            pl.BlockSpec((d_in, d_out), lambda i: (0, 0)),
        ],
        out_specs=[
            pl.BlockSpec((blk, d_out), lambda i: (i, 0)),
            pl.BlockSpec((blk, d_out), lambda i: (i, 0)),
        ],
        out_shape=[
            jax.ShapeDtypeStruct((n, d_out), jnp.float32),
            jax.ShapeDtypeStruct((n, d_out), jnp.float32),
        ],
    )(user_x, item_x, user_weight, item_weight)


def _sc_body(xwu_hbm, xwi_hbm, rows_hbm, cols_hbm, vals_hbm,
             outu_hbm, outi_hbm,
             isrc0, isrc1, idst0, idst1, valb0, valb1,
             msga_v, msgb_v, msgc_v, acc_sh,
             gsem0, gsem1, gsem2, psem0, psem1):
    msg = (msga_v, msgb_v, msgc_v)
    gsem = (gsem0, gsem1, gsem2)
    isrc = (isrc0, isrc1)
    idst = (idst0, idst1)
    valb = (valb0, valb1)
    psem = (psem0, psem1)
    cid = lax.axis_index("c")
    sid = lax.axis_index("s")

    @pl.loop(0, ROW_BLK)
    def _(r):
        for b in range(D // LANES):
            msga_v.at[pl.ds(r, 1), pl.ds(b * LANES, LANES)][...] = (
                jnp.zeros((1, LANES), jnp.float32))
    for k in range(RPS // ROW_BLK):
        pltpu.sync_copy(msga_v.at[pl.ds(0, ROW_BLK)],
                        acc_sh.at[pl.ds(sid * RPS + k * ROW_BLK, ROW_BLK)])
    plsc.subcore_barrier()

    def scale(msg_v, vref, k):
        @pl.loop(0, CHUNK, step=LANES)
        def _(j0):
            val16 = vref[k, pl.ds(j0, LANES)]
            for u in range(LANES):
                srow = jnp.full((1, LANES), val16[u], jnp.float32)
                for b in range(D // LANES):
                    slc = (pl.ds(j0 + u, 1), pl.ds(b * LANES, LANES))
                    msg_v.at[slc][...] = msg_v.at[slc][...] * srow

    def prefetch(src_hbm, dst_hbm, t, s):
        pltpu.async_copy(src_hbm.at[sid, t], isrc[s], psem[s])
        pltpu.async_copy(dst_hbm.at[sid, t], idst[s], psem[s])
        pltpu.async_copy(vals_hbm.at[sid, t], valb[s], psem[s])

    def wait_isrc(src_hbm, t, s):
        pltpu.make_async_copy(src_hbm.at[sid, t], isrc[s], psem[s]).wait()

    def wait_dstval(dst_hbm, t, s):
        pltpu.make_async_copy(dst_hbm.at[sid, t], idst[s], psem[s]).wait()
        pltpu.make_async_copy(vals_hbm.at[sid, t], valb[s], psem[s]).wait()

    def run_batch(table_hbm, src_hbm, dst_hbm, t, t_next, s):
        o = (s + 1) % 2
        wait_dstval(dst_hbm, t, s)

        @pl.when(t_next < NVB)
        def _():
            prefetch(src_hbm, dst_hbm, t_next, o)
        for k in range(VBATCH):
            b = k % 3
            pltpu.make_async_copy(table_hbm.at[isrc[s].at[k]], msg[b],
                                  gsem[b]).wait()
            scale(msg[b], valb[s], k)
            if k + 2 < VBATCH:
                pltpu.async_copy(table_hbm.at[isrc[s].at[k + 2]],
                                 msg[(k + 2) % 3], gsem[(k + 2) % 3])
            else:
                kn = k + 2 - VBATCH
                @pl.when(t_next < NVB)
                def _():
                    if kn == 0:
                        wait_isrc(src_hbm, t_next, o)
                    pltpu.async_copy(table_hbm.at[isrc[o].at[kn]],
                                     msg[(k + 2) % 3], gsem[(k + 2) % 3])
            pltpu.sync_copy(msg[b], acc_sh.at[idst[s].at[k]], add=True)

    def accumulate(table_hbm, src_hbm, dst_hbm):
        prefetch(src_hbm, dst_hbm, 0, 0)
        wait_isrc(src_hbm, 0, 0)
        pltpu.async_copy(table_hbm.at[isrc[0].at[0]], msg[0], gsem[0])
        pltpu.async_copy(table_hbm.at[isrc[0].at[1]], msg[1], gsem[1])

        @pl.loop(0, (NVB + 1) // 2)
        def _(u):
            run_batch(table_hbm, src_hbm, dst_hbm, 2 * u, 2 * u + 1, 0)

            @pl.when(2 * u + 1 < NVB)
            def _():
                run_batch(table_hbm, src_hbm, dst_hbm, 2 * u + 1,
                          2 * u + 2, 1)

    @pl.when(cid == 0)
    def _():
        accumulate(xwi_hbm, cols_hbm, rows_hbm)

    @pl.when(cid == 1)
    def _():
        accumulate(xwu_hbm, rows_hbm, cols_hbm)

    plsc.subcore_barrier()

    def writeback(out_hbm):
        for k in range(RPS // ROW_BLK):
            r0 = sid * RPS + k * ROW_BLK
            pltpu.sync_copy(acc_sh.at[pl.ds(r0, ROW_BLK)],
                            msga_v.at[pl.ds(0, ROW_BLK)])

            @pl.loop(0, ROW_BLK)
            def _(r):
                for b in range(D // LANES):
                    slc = (pl.ds(r, 1), pl.ds(b * LANES, LANES))
                    msga_v.at[slc][...] = jnp.maximum(msga_v.at[slc][...],
                                                      0.0)
            pltpu.sync_copy(msga_v.at[pl.ds(0, ROW_BLK)],
                            out_hbm.at[pl.ds(r0, ROW_BLK)])

    @pl.when(cid == 0)
    def _():
        writeback(outu_hbm)

    @pl.when(cid == 1)
    def _():
        writeback(outi_hbm)


def kernel(user_x, item_x, ui_indices, ui_values, user_weight, item_weight):
    xw_user, xw_item = _project(user_x, item_x, user_weight, item_weight)
    pad = E_PAD - E
    rows = jnp.concatenate(
        [ui_indices[0].astype(jnp.int32), jnp.zeros((pad,), jnp.int32)])
    cols = jnp.concatenate(
        [ui_indices[1].astype(jnp.int32), jnp.zeros((pad,), jnp.int32)])
    vals = jnp.concatenate([ui_values, jnp.zeros((pad,), jnp.float32)])
    rows4 = rows.reshape(NSC, NVB, VBATCH, CHUNK)
    cols4 = cols.reshape(NSC, NVB, VBATCH, CHUNK)
    vals4 = vals.reshape(NSC, NVB, VBATCH, CHUNK)

    mesh = plsc.VectorSubcoreMesh(core_axis_name="c", subcore_axis_name="s")
    sc_fn = pl.kernel(
        _sc_body,
        out_type=[
            jax.ShapeDtypeStruct((N_PAD, D), jnp.float32),
            jax.ShapeDtypeStruct((N_PAD, D), jnp.float32),
        ],
        mesh=mesh,
        scratch_types=[
            pltpu.VMEM((VBATCH, CHUNK), jnp.int32),
            pltpu.VMEM((VBATCH, CHUNK), jnp.int32),
            pltpu.VMEM((VBATCH, CHUNK), jnp.int32),
            pltpu.VMEM((VBATCH, CHUNK), jnp.int32),
            pltpu.VMEM((VBATCH, CHUNK), jnp.float32),
            pltpu.VMEM((VBATCH, CHUNK), jnp.float32),
            pltpu.VMEM((CHUNK, D), jnp.float32),
            pltpu.VMEM((CHUNK, D), jnp.float32),
            pltpu.VMEM((CHUNK, D), jnp.float32),
            pltpu.VMEM_SHARED((N_PAD, D), jnp.float32),
            pltpu.SemaphoreType.DMA,
            pltpu.SemaphoreType.DMA,
            pltpu.SemaphoreType.DMA,
            pltpu.SemaphoreType.DMA,
            pltpu.SemaphoreType.DMA,
        ],
    )
    out_user, out_item = sc_fn(xw_user, xw_item, rows4, cols4, vals4)
    return (out_user[:N_NODES], out_item[:N_NODES])
